# baseline (speedup 1.0000x reference)
import math

import jax
import jax.numpy as jnp
import numpy as np
from jax.experimental import pallas as pl
from jax.experimental.pallas import tpu as pltpu

_B, _L, _H, _D = 4, 2048, 16, 64
_U = 5 * int(np.ceil(np.log(_L)))
_TQ = 256
_TK = 256
_NEG = -1e30


def _rotl32(x, d):
    d = np.uint32(d)
    return ((x << d) | (x >> (np.uint32(32) - d))).astype(np.uint32)


def _threefry2x32(k0, k1, x0, x1):
    rot = [np.uint32(r) for r in (13, 15, 26, 6, 17, 29, 16, 24)]
    ks0, ks1 = np.uint32(k0), np.uint32(k1)
    ks2 = np.uint32(ks0 ^ ks1 ^ np.uint32(0x1BD11BDA))
    x0 = (x0 + ks0).astype(np.uint32)
    x1 = (x1 + ks1).astype(np.uint32)
    inject = [(ks1, ks2), (ks2, ks0), (ks0, ks1), (ks1, ks2), (ks2, ks0)]
    rounds = [rot[:4], rot[4:], rot[:4], rot[4:], rot[:4]]
    for r in range(5):
        for d in rounds[r]:
            x0 = (x0 + x1).astype(np.uint32)
            x1 = (_rotl32(x1, d) ^ x0).astype(np.uint32)
        a, b = inject[r]
        x0 = (x0 + a).astype(np.uint32)
        x1 = (x1 + b + np.uint32(r + 1)).astype(np.uint32)
    return x0, x1


def _np_random_bits(k0, k1, n):
    b1, b2 = _threefry2x32(
        k0, k1, np.zeros(n, np.uint32), np.arange(n, dtype=np.uint32)
    )
    return b1 ^ b2


def _np_randint(seed, shape, lo, hi):
    o0, o1 = _threefry2x32(
        0, seed, np.zeros(2, np.uint32), np.arange(2, dtype=np.uint32)
    )
    n = int(np.prod(shape))
    hb = _np_random_bits(o0[0], o1[0], n)
    lb = _np_random_bits(o0[1], o1[1], n)
    span = int(hi - lo)
    mult = np.uint32(pow(65536 % span, 2, span))
    val = ((hb % np.uint32(span)) * mult + (lb % np.uint32(span))) % np.uint32(span)
    return (np.int32(lo) + val.astype(np.int32)).reshape(shape)


def _count_matrix_T() -> np.ndarray:
    idx = _np_randint(1234, (_L, _U), 0, _L)
    c = np.zeros((_L, _L), dtype=np.float32)
    np.add.at(c, (np.arange(_L)[:, None], idx), 1.0)
    return np.ascontiguousarray(c.T)


_C_T_HOST = _count_matrix_T()


def _body(c_ref, q_ref, k_ref, v_ref, o_ref, m_scr):
    scale = 1.0 / math.sqrt(_D)

    def qb_body(qi, m_line):
        q_blk = q_ref[pl.ds(qi * _TQ, _TQ), :]

        def kb_body(ki, carry):
            rmax, rsum = carry
            k_blk = k_ref[pl.ds(ki * _TK, _TK), :]
            s = jax.lax.dot_general(
                k_blk, q_blk, (((1,), (1,)), ((), ())),
                preferred_element_type=jnp.float32,
            )
            c = c_ref[pl.ds(ki * _TK, _TK), pl.ds(qi * _TQ, _TQ)]
            masked = jnp.where(c > 0, s, _NEG)
            rmax = jnp.maximum(rmax, jnp.max(masked, axis=0, keepdims=True))
            rsum = rsum + jnp.sum(c * s, axis=0, keepdims=True)
            return rmax, rsum

        rmax, rsum = jax.lax.fori_loop(
            0, _L // _TK, kb_body,
            (jnp.full((1, _TQ), _NEG, jnp.float32),
             jnp.zeros((1, _TQ), jnp.float32)),
        )
        m_scr[:, pl.ds(qi * _TQ, _TQ)] = rmax - rsum * (1.0 / _L)
        return 0

    jax.lax.fori_loop(0, _L // _TQ, qb_body, 0)
    m_line = m_scr[:, :]

    qiota = jax.lax.broadcasted_iota(jnp.int32, (1, _L), 1)

    def top_body(i, carry):
        m, sel = carry
        mx = jnp.max(m)
        cand = jnp.where(m == mx, qiota, jnp.int32(_L))
        amin = jnp.min(cand)
        hit = qiota == amin
        sel = jnp.where(hit, i, sel)
        m = jnp.where(hit, _NEG, m)
        return m, sel

    _, sel = jax.lax.fori_loop(
        0, _U, top_body,
        (m_line, jnp.full((1, _L), -1, jnp.int32)),
    )

    riota = jax.lax.broadcasted_iota(jnp.int32, (_U, _L), 0)
    onehot = (riota == sel).astype(jnp.float32)

    q_red = jax.lax.dot_general(
        onehot, q_ref[:, :], (((1,), (0,)), ((), ())),
        preferred_element_type=jnp.float32,
    )
    scores = jax.lax.dot_general(
        q_red, k_ref[:, :], (((1,), (1,)), ((), ())),
        preferred_element_type=jnp.float32,
    ) * scale
    smax = jnp.max(scores, axis=1, keepdims=True)
    e = jnp.exp(scores - smax)
    attn = e / jnp.sum(e, axis=1, keepdims=True)
    upd = jax.lax.dot_general(
        attn, v_ref[:, :], (((1,), (0,)), ((), ())),
        preferred_element_type=jnp.float32,
    )

    vmean = jnp.mean(v_ref[:, :], axis=0, keepdims=True)
    ctx = jax.lax.dot_general(
        onehot, upd - vmean, (((0,), (0,)), ((), ())),
        preferred_element_type=jnp.float32,
    ) + vmean
    o_ref[:, :] = ctx


def kernel(queries, keys, values, attn_mask):
    del attn_mask
    B, L, H, D = queries.shape
    q = jnp.transpose(queries, (0, 2, 1, 3)).reshape(B * H, L, D)
    k = jnp.transpose(keys, (0, 2, 1, 3)).reshape(B * H, L, D)
    v = jnp.transpose(values, (0, 2, 1, 3)).reshape(B * H, L, D)
    c_t = jnp.asarray(_C_T_HOST)

    out = pl.pallas_call(
        _body,
        grid=(B * H,),
        in_specs=[
            pl.BlockSpec((_L, _L), lambda i: (0, 0)),
            pl.BlockSpec((None, _L, _D), lambda i: (i, 0, 0)),
            pl.BlockSpec((None, _L, _D), lambda i: (i, 0, 0)),
            pl.BlockSpec((None, _L, _D), lambda i: (i, 0, 0)),
        ],
        out_specs=pl.BlockSpec((None, _L, _D), lambda i: (i, 0, 0)),
        out_shape=jax.ShapeDtypeStruct((B * H, L, D), jnp.float32),
        scratch_shapes=[pltpu.VMEM((1, _L), jnp.float32)],
        compiler_params=pltpu.CompilerParams(
            dimension_semantics=("arbitrary",),
        ),
    )(c_t, q, k, v)
    return out.reshape(B, H, L, D)


# speedup vs baseline: 3.4391x; 3.4391x over previous
import math

import jax
import jax.numpy as jnp
import numpy as np
from jax.experimental import pallas as pl
from jax.experimental.pallas import tpu as pltpu

_B, _L, _H, _D = 4, 2048, 16, 64
_U = 5 * int(np.ceil(np.log(_L)))
_TQ = 256
_NEG = -1e30


def _rotl32(x, d):
    d = np.uint32(d)
    return ((x << d) | (x >> (np.uint32(32) - d))).astype(np.uint32)


def _threefry2x32(k0, k1, x0, x1):
    rot = [np.uint32(r) for r in (13, 15, 26, 6, 17, 29, 16, 24)]
    ks0, ks1 = np.uint32(k0), np.uint32(k1)
    ks2 = np.uint32(ks0 ^ ks1 ^ np.uint32(0x1BD11BDA))
    x0 = (x0 + ks0).astype(np.uint32)
    x1 = (x1 + ks1).astype(np.uint32)
    inject = [(ks1, ks2), (ks2, ks0), (ks0, ks1), (ks1, ks2), (ks2, ks0)]
    rounds = [rot[:4], rot[4:], rot[:4], rot[4:], rot[:4]]
    for r in range(5):
        for d in rounds[r]:
            x0 = (x0 + x1).astype(np.uint32)
            x1 = (_rotl32(x1, d) ^ x0).astype(np.uint32)
        a, b = inject[r]
        x0 = (x0 + a).astype(np.uint32)
        x1 = (x1 + b + np.uint32(r + 1)).astype(np.uint32)
    return x0, x1


def _np_random_bits(k0, k1, n):
    b1, b2 = _threefry2x32(
        k0, k1, np.zeros(n, np.uint32), np.arange(n, dtype=np.uint32)
    )
    return b1 ^ b2


def _np_randint(seed, shape, lo, hi):
    o0, o1 = _threefry2x32(
        0, seed, np.zeros(2, np.uint32), np.arange(2, dtype=np.uint32)
    )
    n = int(np.prod(shape))
    hb = _np_random_bits(o0[0], o1[0], n)
    lb = _np_random_bits(o0[1], o1[1], n)
    span = int(hi - lo)
    mult = np.uint32(pow(65536 % span, 2, span))
    val = ((hb % np.uint32(span)) * mult + (lb % np.uint32(span))) % np.uint32(span)
    return (np.int32(lo) + val.astype(np.int32)).reshape(shape)


def _count_matrix_T() -> np.ndarray:
    idx = _np_randint(1234, (_L, _U), 0, _L)
    c = np.zeros((_L, _L), dtype=np.float32)
    np.add.at(c, (np.arange(_L)[:, None], idx), 1.0)
    return np.ascontiguousarray(c.T)


_C_T_HOST = _count_matrix_T()


def _p1_stats(c_ref, q_ref, k_ref, m_ref):
    def qb_body(qi, _):
        q_blk = q_ref[pl.ds(qi * _TQ, _TQ), :]
        st = jax.lax.dot_general(
            k_ref[:, :], q_blk, (((1,), (1,)), ((), ())),
            preferred_element_type=jnp.float32,
        )
        c = c_ref[:, pl.ds(qi * _TQ, _TQ)]
        mx = jnp.max(jnp.where(c > 0, st, _NEG), axis=0, keepdims=True)
        sm = jnp.sum(c * st, axis=0, keepdims=True)
        m_ref[:, pl.ds(qi * _TQ, _TQ)] = mx - sm * (1.0 / _L)
        return 0

    jax.lax.fori_loop(0, _L // _TQ, qb_body, 0)


def _p2_topk(m_ref, sel_ref):
    m = m_ref[:, 0, :]
    qiota = jax.lax.broadcasted_iota(jnp.int32, (_B * _H, _L), 1)

    def top_body(i, carry):
        m, sel = carry
        mx = jnp.max(m, axis=1, keepdims=True)
        cand = jnp.where(m == mx, qiota, jnp.int32(_L))
        amin = jnp.min(cand, axis=1, keepdims=True)
        hit = qiota == amin
        sel = jnp.where(hit, i, sel)
        m = jnp.where(hit, _NEG, m)
        return m, sel

    _, sel = jax.lax.fori_loop(
        0, _U, top_body,
        (m, jnp.full((_B * _H, _L), -1, jnp.int32)),
    )
    sel_ref[:, 0, :] = sel


def _p3_attend(sel_ref, q_ref, k_ref, v_ref, o_ref):
    scale = 1.0 / math.sqrt(_D)
    sel = sel_ref[:, :]
    riota = jax.lax.broadcasted_iota(jnp.int32, (_U, _L), 0)
    onehot = (riota == sel).astype(jnp.float32)

    q_red = jax.lax.dot_general(
        onehot, q_ref[:, :], (((1,), (0,)), ((), ())),
        preferred_element_type=jnp.float32,
    )
    scores = jax.lax.dot_general(
        q_red, k_ref[:, :], (((1,), (1,)), ((), ())),
        preferred_element_type=jnp.float32,
    ) * scale
    smax = jnp.max(scores, axis=1, keepdims=True)
    e = jnp.exp(scores - smax)
    attn = e / jnp.sum(e, axis=1, keepdims=True)
    upd = jax.lax.dot_general(
        attn, v_ref[:, :], (((1,), (0,)), ((), ())),
        preferred_element_type=jnp.float32,
    )

    vmean = jnp.mean(v_ref[:, :], axis=0, keepdims=True)
    ctx = jax.lax.dot_general(
        onehot, upd - vmean, (((0,), (0,)), ((), ())),
        preferred_element_type=jnp.float32,
    ) + vmean
    o_ref[:, :] = ctx


def kernel(queries, keys, values, attn_mask):
    del attn_mask
    B, L, H, D = queries.shape
    BH = B * H
    q = jnp.transpose(queries, (0, 2, 1, 3)).reshape(BH, L, D)
    k = jnp.transpose(keys, (0, 2, 1, 3)).reshape(BH, L, D)
    v = jnp.transpose(values, (0, 2, 1, 3)).reshape(BH, L, D)
    c_t = jnp.asarray(_C_T_HOST)

    m = pl.pallas_call(
        _p1_stats,
        grid=(BH,),
        in_specs=[
            pl.BlockSpec((_L, _L), lambda i: (0, 0)),
            pl.BlockSpec((None, _L, _D), lambda i: (i, 0, 0)),
            pl.BlockSpec((None, _L, _D), lambda i: (i, 0, 0)),
        ],
        out_specs=pl.BlockSpec((None, 1, _L), lambda i: (i, 0, 0)),
        out_shape=jax.ShapeDtypeStruct((BH, 1, _L), jnp.float32),
        compiler_params=pltpu.CompilerParams(
            dimension_semantics=("arbitrary",),
        ),
    )(c_t, q, k)

    sel = pl.pallas_call(
        _p2_topk,
        in_specs=[pl.BlockSpec((BH, 1, _L), lambda: (0, 0, 0))],
        out_specs=pl.BlockSpec((BH, 1, _L), lambda: (0, 0, 0)),
        out_shape=jax.ShapeDtypeStruct((BH, 1, _L), jnp.int32),
    )(m)

    out = pl.pallas_call(
        _p3_attend,
        grid=(BH,),
        in_specs=[
            pl.BlockSpec((None, 1, _L), lambda i: (i, 0, 0)),
            pl.BlockSpec((None, _L, _D), lambda i: (i, 0, 0)),
            pl.BlockSpec((None, _L, _D), lambda i: (i, 0, 0)),
            pl.BlockSpec((None, _L, _D), lambda i: (i, 0, 0)),
        ],
        out_specs=pl.BlockSpec((None, _L, _D), lambda i: (i, 0, 0)),
        out_shape=jax.ShapeDtypeStruct((BH, L, D), jnp.float32),
        compiler_params=pltpu.CompilerParams(
            dimension_semantics=("arbitrary",),
        ),
    )(sel, q, k, v)
    return out.reshape(B, H, L, D)
